# Initial kernel scaffold; baseline (speedup 1.0000x reference)
import jax
import jax.numpy as jnp
from jax.experimental import pallas as pl


def kernel(x, adj, W_l, b_l, W_r, b_r, att, bias, ln_w, ln_b):
    raise NotImplementedError("write your pallas kernel here")


# speedup vs baseline: 557.6013x; 557.6013x over previous
import functools

import jax
import jax.numpy as jnp
from jax.experimental import pallas as pl
from jax.experimental.pallas import tpu as pltpu

_B, _N, _IN, _H, _C = 4, 512, 128, 4, 32
_HC = _H * _C
_SBLK = 32


def _gat_kernel(x_ref, xT_ref, adjT_ref, wl_ref, wlT_ref, blr_ref, blc_ref,
                wr_ref, brr_ref, att_ref, bias_ref, lnw_ref,
                lnb_ref, out_ref, p_scr, xr_scr):
    x = x_ref[0]
    xT = xT_ref[0]

    xl = jnp.dot(x, wl_ref[...], preferred_element_type=jnp.float32) + blr_ref[...]
    xr_scr[...] = jnp.dot(x, wr_ref[...], preferred_element_type=jnp.float32) + brr_ref[...]
    xlT = jnp.dot(wlT_ref[...], xT, preferred_element_type=jnp.float32) + blc_ref[...]

    att = att_ref[...]
    att04 = att * 0.4

    outs = []
    for h in range(_H):
        xl_h = xl[:, h * _C:(h + 1) * _C]
        xlT_h = xlT[h * _C:(h + 1) * _C, :]
        al6 = 0.6 * jnp.dot(att[h:h + 1, :], xlT_h,
                            preferred_element_type=jnp.float32)

        def strip_body(s, _, h=h, xlT_h=xlT_h, al6=al6):
            js = s * _SBLK
            xr_s = xr_scr[pl.ds(js, _SBLK), h * _C:(h + 1) * _C]
            acc = jnp.zeros((_SBLK, _N), jnp.float32)
            for c in range(_C):
                z = xr_s[:, c:c + 1] + xlT_h[c:c + 1, :]
                acc = acc + att04[h, c] * jnp.abs(z)
            ar6_s = 0.6 * jnp.sum(xr_s * att[h:h + 1, :], axis=1,
                                  keepdims=True)
            score = acc + (al6 + ar6_s)
            jrow = js + jax.lax.broadcasted_iota(jnp.int32, (_SBLK, _N), 0)
            icol = jax.lax.broadcasted_iota(jnp.int32, (_SBLK, _N), 1)
            adjt = adjT_ref[0, pl.ds(js, _SBLK), :]
            mask = (jrow == icol) | (adjt != 0.0)
            sm = jnp.where(mask, score, -1e30)
            m = jnp.max(sm, axis=1, keepdims=True)
            e = jnp.where(mask, jnp.exp(sm - m), 0.0)
            den = jnp.sum(e, axis=1, keepdims=True)
            p = e * (1.0 / (den + 1e-16))
            p_scr[pl.ds(js, _SBLK), :] = p
            return 0

        jax.lax.fori_loop(0, _N // _SBLK, strip_body, 0)
        outs.append(jnp.dot(p_scr[...], xl_h, preferred_element_type=jnp.float32))

    o = jnp.concatenate(outs, axis=1) + bias_ref[...]
    mu = jnp.mean(o, axis=1, keepdims=True)
    d = o - mu
    var = jnp.mean(d * d, axis=1, keepdims=True)
    out_ref[0] = d * jax.lax.rsqrt(var + 1e-5) * lnw_ref[...] + lnb_ref[...]


@jax.jit
def kernel(x, adj, W_l, b_l, W_r, b_r, att, bias, ln_w, ln_b):
    xT = jnp.swapaxes(x, 1, 2)
    adjT = jnp.swapaxes(adj, 1, 2)
    wlT = W_l.T
    att_p = jnp.zeros((8, _C), att.dtype).at[:_H].set(att)
    blr = b_l.reshape(1, _HC)
    blc = b_l.reshape(_HC, 1)
    brr = b_r.reshape(1, _HC)
    bias2 = bias.reshape(1, _HC)
    lnw2 = ln_w.reshape(1, _HC)
    lnb2 = ln_b.reshape(1, _HC)

    full = lambda *shape: pl.BlockSpec(shape, lambda b: (0,) * len(shape))
    grid_spec = pltpu.PrefetchScalarGridSpec(
        num_scalar_prefetch=0,
        grid=(_B,),
        in_specs=[
            pl.BlockSpec((1, _N, _IN), lambda b: (b, 0, 0)),
            pl.BlockSpec((1, _IN, _N), lambda b: (b, 0, 0)),
            pl.BlockSpec((1, _N, _N), lambda b: (b, 0, 0)),
            full(_IN, _HC),
            full(_HC, _IN),
            full(1, _HC),
            full(_HC, 1),
            full(_IN, _HC),
            full(1, _HC),
            full(8, _C),
            full(1, _HC),
            full(1, _HC),
            full(1, _HC),
        ],
        out_specs=pl.BlockSpec((1, _N, _HC), lambda b: (b, 0, 0)),
        scratch_shapes=[pltpu.VMEM((_N, _N), jnp.float32),
                        pltpu.VMEM((_N, _HC), jnp.float32)],
    )
    return pl.pallas_call(
        _gat_kernel,
        grid_spec=grid_spec,
        out_shape=jax.ShapeDtypeStruct((_B, _N, _HC), jnp.float32),
        compiler_params=pltpu.CompilerParams(
            dimension_semantics=("arbitrary",),
        ),
    )(x, xT, adjT, W_l, wlT, blr, blc, W_r, brr, att_p, bias2,
      lnw2, lnb2)
